# Initial kernel scaffold; baseline (speedup 1.0000x reference)
import jax
import jax.numpy as jnp
from jax.experimental import pallas as pl


def kernel(x, edge_index, batch, W_proj, b_proj, W1, b1, W2, b2, W3, b3, W_out, b_out):
    raise NotImplementedError("write your pallas kernel here")


# speedup vs baseline: 15.9123x; 15.9123x over previous
import functools
import jax
import jax.numpy as jnp
from jax import lax
from jax.experimental import pallas as pl
from jax.experimental.pallas import tpu as pltpu
from jax.experimental.pallas import tpu_sc as plsc

NC = 2
NS = 16
NW = NC * NS
CH = 128

F32 = jnp.float32


def _mesh():
    return plsc.VectorSubcoreMesh(
        core_axis_name="c", subcore_axis_name="s", num_cores=NC, num_subcores=NS
    )


def _sc_deg_body(cpt, rps, dstc, ones_h, zeros_h, out, dst_v, ones_v, acc):
    c = lax.axis_index("c")
    s = lax.axis_index("s")
    wid = s * NC + c
    off = pl.multiple_of(s * rps, 8)
    pltpu.sync_copy(dstc.at[wid], dst_v)
    pltpu.sync_copy(ones_h, ones_v)
    pltpu.sync_copy(zeros_h.at[pl.ds(off, rps)], acc.at[pl.ds(off, rps)])
    plsc.subcore_barrier()

    def body(j, carry):
        pltpu.sync_copy(ones_v, acc.at[dst_v.at[j]], add=True)
        return carry

    lax.fori_loop(0, cpt, body, 0)
    plsc.subcore_barrier()
    pltpu.sync_copy(acc.at[pl.ds(off, rps)], out.at[c, pl.ds(off, rps)])


def _sc_prop_body(cpt, rps, g_h, srcc, dstc, zeros_h, out,
                  src_v, dst_v, buf, acc, sem):
    c = lax.axis_index("c")
    s = lax.axis_index("s")
    wid = s * NC + c
    off = pl.multiple_of(s * rps, 8)
    pltpu.sync_copy(srcc.at[wid], src_v)
    pltpu.sync_copy(dstc.at[wid], dst_v)
    pltpu.sync_copy(zeros_h.at[pl.ds(off, rps)], acc.at[pl.ds(off, rps)])
    plsc.subcore_barrier()

    def body(j, carry):
        pltpu.async_copy(g_h.at[src_v.at[j]], buf, sem).wait()
        pltpu.sync_copy(buf, acc.at[dst_v.at[j]], add=True)
        return carry

    lax.fori_loop(0, cpt, body, 0)
    plsc.subcore_barrier()
    pltpu.sync_copy(acc.at[pl.ds(off, rps)], out.at[c, pl.ds(off, rps)])


def _tc0_body(x_ref, w_ref, b_ref, o_ref):
    o_ref[...] = jax.nn.relu(
        jnp.dot(x_ref[...], w_ref[...], preferred_element_type=F32) + b_ref[...]
    )


def _tc1_body(h0_ref, d0_ref, d1_ref, w_ref, g_ref, dis_ref):
    deg = d0_ref[:, :1] + d1_ref[:, :1] + 1.0
    dis = lax.rsqrt(deg)
    dis_ref[...] = dis
    g_ref[...] = dis * jnp.dot(h0_ref[...], w_ref[...], preferred_element_type=F32)


def _tc2_body(a0_ref, a1_ref, g_ref, dis_ref, w_ref, b_ref, gn_ref):
    dis = dis_ref[...]
    h = jax.nn.relu(dis * (a0_ref[...] + a1_ref[...] + g_ref[...]) + b_ref[...])
    gn_ref[...] = dis * jnp.dot(h, w_ref[...], preferred_element_type=F32)


def _tc4_body(nblk, a0_ref, a1_ref, g_ref, dis_ref, b_ref, bat_ref, wo_ref,
              bo_ref, o_ref, psum, csum):
    i = pl.program_id(0)

    @pl.when(i == 0)
    def _():
        psum[...] = jnp.zeros_like(psum)
        csum[...] = jnp.zeros_like(csum)

    dis = dis_ref[...]
    h = jax.nn.relu(dis * (a0_ref[...] + a1_ref[...] + g_ref[...]) + b_ref[...])
    r = h.shape[0]
    oh = (lax.broadcasted_iota(jnp.int32, (16, r), 0) == bat_ref[0]).astype(F32)
    psum[...] += jnp.dot(oh, h, preferred_element_type=F32)
    csum[...] += jnp.sum(oh, axis=1, keepdims=True)

    @pl.when(i == nblk - 1)
    def _():
        pooled = psum[...] / jnp.maximum(csum[...], 1.0)
        o_ref[...] = jnp.dot(pooled, wo_ref[...], preferred_element_type=F32) + bo_ref[...]


def kernel(x, edge_index, batch, W_proj, b_proj, W1, b1, W2, b2, W3, b3,
           W_out, b_out):
    n, f_in = x.shape
    h = W1.shape[0]
    out_dim = W_out.shape[1]
    e = edge_index.shape[1]

    cpt = -(-e // (NW * CH))
    epad = NW * CH * cpt
    accr = -(-(n + 1) // (NS * 8)) * (NS * 8)
    rps = accr // NS

    src = edge_index[0]
    dst = edge_index[1]
    pad = epad - e
    srcc = jnp.concatenate([src, jnp.zeros((pad,), jnp.int32)]).reshape(NW, cpt, CH)
    dstc = jnp.concatenate([dst, jnp.full((pad,), n, jnp.int32)]).reshape(NW, cpt, CH)
    zeros16 = jnp.zeros((accr, 16), F32)
    zeros64 = jnp.zeros((accr, h), F32)
    ones16 = jnp.ones((CH, 16), F32)

    sc_deg = pl.kernel(
        functools.partial(_sc_deg_body, cpt, rps),
        out_type=jax.ShapeDtypeStruct((NC, accr, 16), F32),
        mesh=_mesh(),
        compiler_params=pltpu.CompilerParams(use_tc_tiling_on_sc=False),
        scratch_types=[
            pltpu.VMEM((cpt, CH), jnp.int32),
            pltpu.VMEM((CH, 16), F32),
            pltpu.VMEM_SHARED((accr, 16), F32),
        ],
    )
    sc_prop = pl.kernel(
        functools.partial(_sc_prop_body, cpt, rps),
        out_type=jax.ShapeDtypeStruct((NC, accr, h), F32),
        mesh=_mesh(),
        compiler_params=pltpu.CompilerParams(use_tc_tiling_on_sc=False),
        scratch_types=[
            pltpu.VMEM((cpt, CH), jnp.int32),
            pltpu.VMEM((cpt, CH), jnp.int32),
            pltpu.VMEM((CH, h), F32),
            pltpu.VMEM_SHARED((accr, h), F32),
            pltpu.SemaphoreType.DMA,
        ],
    )

    nblk = 10
    r = n // nblk
    row_spec = lambda w: pl.BlockSpec((r, w), lambda i: (i, 0))
    full_spec = lambda a, b: pl.BlockSpec((a, b), lambda i: (0, 0))

    tc0 = pl.pallas_call(
        _tc0_body,
        grid=(nblk,),
        in_specs=[row_spec(f_in), full_spec(f_in, h), full_spec(1, h)],
        out_specs=row_spec(h),
        out_shape=jax.ShapeDtypeStruct((n, h), F32),
    )
    tc1 = pl.pallas_call(
        _tc1_body,
        grid=(nblk,),
        in_specs=[row_spec(h), row_spec(16), row_spec(16), full_spec(h, h)],
        out_specs=[row_spec(h), row_spec(1)],
        out_shape=[jax.ShapeDtypeStruct((n, h), F32),
                   jax.ShapeDtypeStruct((n, 1), F32)],
    )
    tc2 = pl.pallas_call(
        _tc2_body,
        grid=(nblk,),
        in_specs=[row_spec(h), row_spec(h), row_spec(h), row_spec(1),
                  full_spec(h, h), full_spec(1, h)],
        out_specs=row_spec(h),
        out_shape=jax.ShapeDtypeStruct((n, h), F32),
    )
    tc4 = pl.pallas_call(
        functools.partial(_tc4_body, nblk),
        grid=(nblk,),
        in_specs=[row_spec(h), row_spec(h), row_spec(h), row_spec(1),
                  full_spec(1, h), pl.BlockSpec((1, 1, r), lambda i: (i, 0, 0)),
                  full_spec(h, out_dim), full_spec(1, out_dim)],
        out_specs=full_spec(16, out_dim),
        out_shape=jax.ShapeDtypeStruct((16, out_dim), F32),
        scratch_shapes=[pltpu.VMEM((16, h), F32), pltpu.VMEM((16, 1), F32)],
    )

    degp = sc_deg(dstc, ones16, zeros16)
    h0 = tc0(x, W_proj, b_proj.reshape(1, h))
    g1, dis = tc1(h0, degp[0, :n, :], degp[1, :n, :], W1)

    acc1 = sc_prop(g1, srcc, dstc, zeros64)
    g2 = tc2(acc1[0, :n, :], acc1[1, :n, :], g1, dis, W2, b1.reshape(1, h))
    acc2 = sc_prop(g2, srcc, dstc, zeros64)
    g3 = tc2(acc2[0, :n, :], acc2[1, :n, :], g2, dis, W3, b2.reshape(1, h))
    acc3 = sc_prop(g3, srcc, dstc, zeros64)

    out = tc4(acc3[0, :n, :], acc3[1, :n, :], g3, dis, b3.reshape(1, h),
              batch.reshape(nblk, 1, r), W_out, b_out.reshape(1, out_dim))
    return out
